# baseline (speedup 1.0000x reference)
import jax
import jax.numpy as jnp
from jax import lax
from jax.experimental import pallas as pl
from jax.experimental.pallas import tpu as pltpu
from jax.experimental.pallas import tpu_sc as plsc

B = 512
NV = 4096
L = 16
NC = 2
NS = 16
NW = NC * NS
BPW = B // NW
LIM = 0.125


def _start(off_hbm, ph_hbm, th_hbm, b, off_buf, ph_buf, th_buf, sem):
    return (
        pltpu.async_copy(off_hbm.at[b], off_buf, sem),
        pltpu.async_copy(ph_hbm.at[b], ph_buf, sem),
        pltpu.async_copy(th_hbm.at[b, 0], th_buf, sem),
    )


def _accumulate(off_buf, ph_buf, th_buf, tsv, wbase, accs):
    ts0, ts1, ts2 = tsv[0], tsv[1], tsv[2]
    t2v = jnp.clip(ts2 - wbase, -LIM, LIM)

    def group(i, accs, ts0=ts0, ts1=ts1, t2v=t2v):
        a = list(accs)
        d = (i >> 4).astype(jnp.float32)
        t0 = jnp.clip(ts0 - (d * 0.125 - 1.0), -LIM, LIM)
        hb = (i & 15).astype(jnp.float32)
        dd = i >> 4
        h0 = i & 15
        for k in range(4):
            t1 = jnp.clip(ts1 - ((hb + float(k)) * 0.125 - 1.0), -LIM, LIM)
            hh = h0 + k
            o0 = off_buf[0, dd, hh, :]
            o1 = off_buf[1, dd, hh, :]
            o2 = off_buf[2, dd, hh, :]
            p0 = ph_buf[0, dd, hh, :]
            p1 = ph_buf[1, dd, hh, :]
            tt = th_buf[dd, hh, :]
            m = jnp.logical_and(p1 > p0, tt >= 0.5)
            s = jnp.abs(o0 - t0) + jnp.abs(o1 - t1) + jnp.abs(o2 - t2v)
            a[k] = a[k] + jnp.where(m, s, 0.0)
            a[4 + k] = a[4 + k] + jnp.where(m, 1.0, 0.0)
        return tuple(a)

    return plsc.parallel_loop(0, NV // L, 4, carry=accs)(group)


def _sc_body(off_hbm, ph_hbm, th_hbm, ts_hbm, out_hbm,
             off0, ph0, th0, off1, ph1, th1, ts_buf, res_buf, sem0, sem1):
    wid = lax.axis_index("s") * NC + lax.axis_index("c")
    base = wid * BPW
    pltpu.sync_copy(ts_hbm.at[pl.ds(base, BPW)], ts_buf)

    wbase = lax.iota(jnp.int32, L).astype(jnp.float32) * 0.125 - 1.0

    zero = jnp.zeros((L,), jnp.float32)
    accs = (zero,) * 8

    slots = ((off0, ph0, th0, sem0), (off1, ph1, th1, sem1))
    pending = [None, None]
    pending[0] = _start(off_hbm, ph_hbm, th_hbm, base, *slots[0])
    for i in range(BPW):
        s = i % 2
        if i + 1 < BPW:
            pending[1 - s] = _start(off_hbm, ph_hbm, th_hbm, base + i + 1,
                                    *slots[1 - s])
        for cp in pending[s]:
            cp.wait()
        obuf, pbuf, tbuf, _ = slots[s]
        accs = _accumulate(obuf, pbuf, tbuf, ts_buf[i, :], wbase, accs)

    tot = (accs[0] + accs[1]) + (accs[2] + accs[3])
    cnt = (accs[4] + accs[5]) + (accs[6] + accs[7])
    res_buf[0, :] = tot
    res_buf[1, :] = cnt
    pltpu.sync_copy(res_buf, out_hbm.at[wid])


def kernel(offsets, target_skeleton, predicted_heatmap, target_heatmap):
    ts = jnp.pad(target_skeleton.reshape(B, 3), ((0, 0), (0, L - 3)))

    mesh = plsc.VectorSubcoreMesh(core_axis_name="c", subcore_axis_name="s")
    f = pl.kernel(
        _sc_body,
        out_type=jax.ShapeDtypeStruct((NW, 2, L), jnp.float32),
        mesh=mesh,
        compiler_params=pltpu.CompilerParams(use_tc_tiling_on_sc=False),
        scratch_types=[
            pltpu.VMEM((3, 16, 16, L), jnp.float32),
            pltpu.VMEM((2, 16, 16, L), jnp.float32),
            pltpu.VMEM((16, 16, L), jnp.float32),
            pltpu.VMEM((3, 16, 16, L), jnp.float32),
            pltpu.VMEM((2, 16, 16, L), jnp.float32),
            pltpu.VMEM((16, 16, L), jnp.float32),
            pltpu.VMEM((BPW, L), jnp.float32),
            pltpu.VMEM((2, L), jnp.float32),
            pltpu.SemaphoreType.DMA,
            pltpu.SemaphoreType.DMA,
        ],
    )
    out = f(offsets, predicted_heatmap, target_heatmap, ts)
    tot = jnp.sum(out[:, 0, :])
    cnt = jnp.sum(out[:, 1, :])
    denom = jnp.maximum(cnt * 3.0, 1.0)
    return jnp.where(cnt > 0, tot / denom, jnp.float32(0.0))


# speedup vs baseline: 13.6531x; 13.6531x over previous
import jax
import jax.numpy as jnp
from jax import lax
from jax.experimental import pallas as pl
from jax.experimental.pallas import tpu as pltpu

B = 512
NV = 4096
VB = 256
GRID = NV // VB
LIM = 0.125


def _tc_body(ts_ref, off_ref, ph_ref, th_ref, out_ref, acc_ref):
    i = pl.program_id(0)

    @pl.when(i == 0)
    def _init():
        acc_ref[...] = jnp.zeros_like(acc_ref)

    vv = lax.broadcasted_iota(jnp.int32, (VB, 1), 0) + i * VB
    b0 = (vv >> 8).astype(jnp.float32) * 0.125 - 1.0
    b1 = ((vv >> 4) & 15).astype(jnp.float32) * 0.125 - 1.0
    b2 = (vv & 15).astype(jnp.float32) * 0.125 - 1.0

    ts0 = ts_ref[0:1, :]
    ts1 = ts_ref[1:2, :]
    ts2 = ts_ref[2:3, :]
    t0 = jnp.clip(ts0 - b0, -LIM, LIM)
    t1 = jnp.clip(ts1 - b1, -LIM, LIM)
    t2 = jnp.clip(ts2 - b2, -LIM, LIM)

    m = jnp.logical_and(ph_ref[1] > ph_ref[0], th_ref[...] >= 0.5)
    mf = m.astype(jnp.float32)
    s = (jnp.abs(off_ref[0] - t0) + jnp.abs(off_ref[1] - t1)
         + jnp.abs(off_ref[2] - t2))
    acc_ref[0:1, :] += jnp.sum(s * mf, axis=0, keepdims=True)
    acc_ref[1:2, :] += jnp.sum(mf, axis=0, keepdims=True)

    @pl.when(i == GRID - 1)
    def _finish():
        tot = jnp.sum(acc_ref[0:1, :])
        cnt = jnp.sum(acc_ref[1:2, :])
        denom = jnp.maximum(cnt * 3.0, 1.0)
        out_ref[0, 0] = jnp.where(cnt > 0, tot / denom, 0.0)


def kernel(offsets, target_skeleton, predicted_heatmap, target_heatmap):
    off_t = jnp.transpose(offsets, (1, 2, 3, 4, 0)).reshape(3, NV, B)
    ph_t = jnp.transpose(predicted_heatmap, (1, 2, 3, 4, 0)).reshape(2, NV, B)
    th_t = jnp.transpose(target_heatmap, (1, 2, 3, 4, 0)).reshape(NV, B)
    ts_t = jnp.transpose(target_skeleton, (2, 1, 0)).reshape(3, B)

    out = pl.pallas_call(
        _tc_body,
        grid=(GRID,),
        in_specs=[
            pl.BlockSpec((3, B), lambda i: (0, 0)),
            pl.BlockSpec((3, VB, B), lambda i: (0, i, 0)),
            pl.BlockSpec((2, VB, B), lambda i: (0, i, 0)),
            pl.BlockSpec((VB, B), lambda i: (i, 0)),
        ],
        out_specs=pl.BlockSpec(memory_space=pltpu.SMEM),
        out_shape=jax.ShapeDtypeStruct((1, 1), jnp.float32),
        scratch_shapes=[pltpu.VMEM((2, B), jnp.float32)],
    )(ts_t, off_t, ph_t, th_t)
    return out[0, 0]


# speedup vs baseline: 13.9175x; 1.0194x over previous
import jax
import jax.numpy as jnp
from jax import lax
from jax.experimental import pallas as pl
from jax.experimental.pallas import tpu as pltpu

B = 512
NV = 4096
VB = 256
GRID = NV // VB
LIM = 0.125


def _tc_body(ts_ref, off_ref, ph_ref, th_ref, out_ref, acc_ref):
    i = pl.program_id(0)

    @pl.when(i == 0)
    def _init():
        acc_ref[...] = jnp.zeros_like(acc_ref)

    rr = lax.broadcasted_iota(jnp.int32, (VB, 1), 0)
    b1 = (rr >> 4).astype(jnp.float32) * 0.125 - 1.0
    b2 = (rr & 15).astype(jnp.float32) * 0.125 - 1.0

    ts0 = ts_ref[0:1, :]
    ts1 = ts_ref[1:2, :]
    ts2 = ts_ref[2:3, :]
    t0 = jnp.clip(ts0 - (i.astype(jnp.float32) * 0.125 - 1.0), -LIM, LIM)
    t1 = jnp.clip(ts1 - b1, -LIM, LIM)
    t2 = jnp.clip(ts2 - b2, -LIM, LIM)

    m = jnp.logical_and(ph_ref[1] > ph_ref[0], th_ref[...] >= 0.5)
    mf = m.astype(jnp.float32)
    s = (jnp.abs(off_ref[0] - t0) + jnp.abs(off_ref[1] - t1)
         + jnp.abs(off_ref[2] - t2))
    acc_ref[0:1, :] += jnp.sum(s * mf, axis=0, keepdims=True)
    acc_ref[1:2, :] += jnp.sum(mf, axis=0, keepdims=True)

    @pl.when(i == GRID - 1)
    def _finish():
        tot = jnp.sum(acc_ref[0:1, :])
        cnt = jnp.sum(acc_ref[1:2, :])
        denom = jnp.maximum(cnt * 3.0, 1.0)
        out_ref[0, 0] = jnp.where(cnt > 0, tot / denom, 0.0)


def kernel(offsets, target_skeleton, predicted_heatmap, target_heatmap):
    off_t = jnp.transpose(offsets, (1, 2, 3, 4, 0)).reshape(3, NV, B)
    ph_t = jnp.transpose(predicted_heatmap, (1, 2, 3, 4, 0)).reshape(2, NV, B)
    th_t = jnp.transpose(target_heatmap, (1, 2, 3, 4, 0)).reshape(NV, B)
    ts_t = jnp.transpose(target_skeleton, (2, 1, 0)).reshape(3, B)

    out = pl.pallas_call(
        _tc_body,
        grid=(GRID,),
        in_specs=[
            pl.BlockSpec((3, B), lambda i: (0, 0)),
            pl.BlockSpec((3, VB, B), lambda i: (0, i, 0)),
            pl.BlockSpec((2, VB, B), lambda i: (0, i, 0)),
            pl.BlockSpec((VB, B), lambda i: (i, 0)),
        ],
        out_specs=pl.BlockSpec(memory_space=pltpu.SMEM),
        out_shape=jax.ShapeDtypeStruct((1, 1), jnp.float32),
        scratch_shapes=[pltpu.VMEM((2, B), jnp.float32)],
    )(ts_t, off_t, ph_t, th_t)
    return out[0, 0]
